# baseline (speedup 1.0000x reference)
import functools

import jax
import jax.numpy as jnp
from jax import lax
from jax.experimental import pallas as pl
from jax.experimental.pallas import tpu as pltpu
from jax.experimental.pallas import tpu_sc as plsc

_A = 256
_S = 32768
_M = 501 * 501 * 2 * 2
_SROW = 256
_BLK = 32

_HALF0 = 502016
_MPAD = 2 * _HALF0
_DUMP = 502016
_SBUF = 502032
_TSLICE = 31376
_ZCH = 7840
_NCH = 16


def _field_body(idx_ref, st_ref, out_ref):
    idx = idx_ref[...]
    i0 = idx // 2004
    rem = idx - i0 * 2004
    i1 = rem // 4
    r4 = rem - i1 * 4
    i2 = r4 // 2
    i3 = r4 - i2 * 2
    xs = i0.astype(jnp.float32) - 250.0
    ys = i1.astype(jnp.float32) - 250.0
    ss = i2.astype(jnp.float32) - 0.5
    ts = i3.astype(jnp.float32) - 0.5
    tq = ts + 0.5

    def body(i, carry):
        acc_q, acc_p = carry
        xi = st_ref[i, 0]
        yi = st_ref[i, 1]
        si = st_ref[i, 2]
        ti = st_ref[i, 3]
        dx = xs - xi
        dy = ys - yi
        dist = jnp.sqrt(dx * dx + dy * dy + 1e-12)
        acc_q = acc_q + (ti + 0.5) / (dist + 1.0)
        sd = dist + jnp.abs(ss - si) + jnp.abs(ts - ti)
        acc_p = acc_p + jnp.where(sd < 1e-3, 1e6, 0.0)
        return acc_q, acc_p

    z = jnp.zeros_like(xs)
    acc_q, acc_p = lax.fori_loop(0, _A, body, (z, z))
    out_ref[...] = acc_p + tq * acc_q


def _compute_field(idx2d, states):
    return pl.pallas_call(
        _field_body,
        grid=(_SROW // _BLK,),
        in_specs=[
            pl.BlockSpec((_BLK, 128), lambda i: (i, 0)),
            pl.BlockSpec(memory_space=pltpu.SMEM),
        ],
        out_specs=pl.BlockSpec((_BLK, 128), lambda i: (i, 0)),
        out_shape=jax.ShapeDtypeStruct((_SROW, 128), jnp.float32),
    )(idx2d, states)


def _sc_scatter(field2d, idx2d):
    mesh = plsc.VectorSubcoreMesh(core_axis_name="c", subcore_axis_name="s")

    @functools.partial(
        pl.kernel,
        mesh=mesh,
        out_type=jax.ShapeDtypeStruct((_MPAD,), jnp.float32),
        scratch_types=(
            [pltpu.VMEM((128,), jnp.int32) for _ in range(_NCH)]
            + [pltpu.VMEM((128,), jnp.float32) for _ in range(_NCH)]
            + [pltpu.VMEM((128,), jnp.int32) for _ in range(_NCH)]
            + [pltpu.VMEM((_ZCH,), jnp.float32)]
            + [pltpu.VMEM_SHARED((_SBUF,), jnp.float32)]
            + [pltpu.SemaphoreType.DMA]
        ),
        compiler_params=pltpu.CompilerParams(needs_layout_passes=False),
    )
    def k(field_hbm, idx_hbm, out_hbm, *scratch):
        idx_refs = scratch[:_NCH]
        val_refs = scratch[_NCH:2 * _NCH]
        rout_refs = scratch[2 * _NCH:3 * _NCH]
        zbuf = scratch[3 * _NCH]
        shared = scratch[3 * _NCH + 1]
        sem = scratch[3 * _NCH + 2]
        c = lax.axis_index("c")
        s = lax.axis_index("s")
        scbase = c * _HALF0

        row0 = s * _NCH
        copies = [
            pltpu.async_copy(idx_hbm.at[row0 + j], idx_refs[j], sem)
            for j in range(_NCH)
        ]
        copies += [
            pltpu.async_copy(field_hbm.at[row0 + j], val_refs[j], sem)
            for j in range(_NCH)
        ]

        def zfill(i, carry):
            zbuf[pl.ds(i * 16, 16)] = jnp.zeros((16,), jnp.float32)
            return carry

        lax.fori_loop(0, _ZCH // 16, zfill, 0)
        sl0 = s * _TSLICE
        for t in range(4):
            pltpu.sync_copy(zbuf, shared.at[pl.ds(sl0 + t * _ZCH, _ZCH)])
        pltpu.sync_copy(
            zbuf.at[pl.ds(0, _TSLICE - 4 * _ZCH)],
            shared.at[pl.ds(sl0 + 4 * _ZCH, _TSLICE - 4 * _ZCH)],
        )

        for cp in copies:
            cp.wait()

        for j in range(_NCH):
            for g in range(8):
                iv = idx_refs[j][pl.ds(g * 16, 16)]
                rel = iv - scbase
                ok = (rel >= 0) & (rel < _HALF0)
                rout_refs[j][pl.ds(g * 16, 16)] = jnp.where(ok, rel, _DUMP)

        plsc.subcore_barrier()

        scats = [
            pltpu.async_copy(val_refs[j], shared.at[rout_refs[j]], sem)
            for j in range(_NCH)
        ]
        for cp in scats:
            cp.wait()

        plsc.subcore_barrier()
        lax.fori_loop(0, _ZCH // 16, zfill, 0)

        obase = scbase + sl0
        for t in range(4):
            pltpu.sync_copy(shared.at[pl.ds(sl0 + t * _ZCH, _ZCH)], zbuf)
            pltpu.sync_copy(zbuf, out_hbm.at[pl.ds(obase + t * _ZCH, _ZCH)])
        tail = _TSLICE - 4 * _ZCH
        pltpu.sync_copy(
            shared.at[pl.ds(sl0 + 4 * _ZCH, tail)], zbuf.at[pl.ds(0, tail)]
        )
        pltpu.sync_copy(
            zbuf.at[pl.ds(0, tail)], out_hbm.at[pl.ds(obase + 4 * _ZCH, tail)]
        )

    return k(field2d, idx2d)


def kernel(states, site_flat_idx):
    idx2d = site_flat_idx.reshape(_SROW, 128)
    field2d = _compute_field(idx2d, states)
    return _sc_scatter(field2d, idx2d)[:_M]


# speedup vs baseline: 1.2117x; 1.2117x over previous
import functools

import jax
import jax.numpy as jnp
from jax import lax
from jax.experimental import pallas as pl
from jax.experimental.pallas import tpu as pltpu
from jax.experimental.pallas import tpu_sc as plsc

_A = 256
_S = 32768
_M = 501 * 501 * 2 * 2
_SROW = 256
_BLK = 32

_HALF0 = 502016
_MPAD = 2 * _HALF0
_DUMP = 502016
_SBUF = 502032
_TSLICE = 31376
_ZCH = 7840
_NCH = 16


def _field_body(idx_ref, st_ref, out_ref):
    idx = idx_ref[...]
    i0 = idx // 2004
    rem = idx - i0 * 2004
    i1 = rem // 4
    r4 = rem - i1 * 4
    i2 = r4 // 2
    i3 = r4 - i2 * 2
    xs = i0.astype(jnp.float32) - 250.0
    ys = i1.astype(jnp.float32) - 250.0
    msp = i2 > 0
    mts = i3 > 0
    tq = i3.astype(jnp.float32)

    def one_state(i, acc_q, acc_p):
        xi = st_ref[i, 0]
        yi = st_ref[i, 1]
        ci = st_ref[i, 2]
        dx = xs - xi
        dy = ys - yi
        d2 = dx * dx + dy * dy
        acc_q = acc_q + ci / (jnp.sqrt(d2) + 1.0)
        thr2 = jnp.where(
            msp,
            jnp.where(mts, st_ref[i, 6], st_ref[i, 5]),
            jnp.where(mts, st_ref[i, 4], st_ref[i, 3]),
        )
        acc_p = acc_p + jnp.where(d2 < thr2, 1e6, 0.0)
        return acc_q, acc_p

    def body(i, carry):
        acc_q, acc_p = carry
        acc_q, acc_p = one_state(2 * i, acc_q, acc_p)
        acc_q, acc_p = one_state(2 * i + 1, acc_q, acc_p)
        return acc_q, acc_p

    z = jnp.zeros_like(xs)
    acc_q, acc_p = lax.fori_loop(0, _A // 2, body, (z, z))
    out_ref[...] = acc_p + tq * acc_q


def _compute_field(idx2d, states):
    xi = states[:, 0]
    yi = states[:, 1]
    si = states[:, 2]
    ti = states[:, 3]
    ci = ti + 0.5
    cols = [xi, yi, ci]
    for a in (-0.5, 0.5):
        for b in (-0.5, 0.5):
            thr = 1e-3 - jnp.abs(si - a) - jnp.abs(ti - b)
            cols.append(jnp.where(thr > 0, thr * thr - 1e-12, -1.0))
    staug = jnp.stack(cols + [jnp.zeros_like(xi)], axis=1)
    return pl.pallas_call(
        _field_body,
        grid=(_SROW // _BLK,),
        in_specs=[
            pl.BlockSpec((_BLK, 128), lambda i: (i, 0)),
            pl.BlockSpec(memory_space=pltpu.SMEM),
        ],
        out_specs=pl.BlockSpec((_BLK, 128), lambda i: (i, 0)),
        out_shape=jax.ShapeDtypeStruct((_SROW, 128), jnp.float32),
    )(idx2d, staug)


def _sc_scatter(field2d, idx2d):
    mesh = plsc.VectorSubcoreMesh(core_axis_name="c", subcore_axis_name="s")

    @functools.partial(
        pl.kernel,
        mesh=mesh,
        out_type=jax.ShapeDtypeStruct((_MPAD,), jnp.float32),
        scratch_types=(
            [pltpu.VMEM((128,), jnp.int32) for _ in range(_NCH)]
            + [pltpu.VMEM((128,), jnp.float32) for _ in range(_NCH)]
            + [pltpu.VMEM((128,), jnp.int32) for _ in range(_NCH)]
            + [pltpu.VMEM((_ZCH,), jnp.float32)]
            + [pltpu.VMEM_SHARED((_SBUF,), jnp.float32)]
            + [pltpu.SemaphoreType.DMA]
        ),
        compiler_params=pltpu.CompilerParams(needs_layout_passes=False),
    )
    def k(field_hbm, idx_hbm, out_hbm, *scratch):
        idx_refs = scratch[:_NCH]
        val_refs = scratch[_NCH:2 * _NCH]
        rout_refs = scratch[2 * _NCH:3 * _NCH]
        zbuf = scratch[3 * _NCH]
        shared = scratch[3 * _NCH + 1]
        sem = scratch[3 * _NCH + 2]
        c = lax.axis_index("c")
        s = lax.axis_index("s")
        scbase = c * _HALF0

        row0 = s * _NCH
        copies = [
            pltpu.async_copy(idx_hbm.at[row0 + j], idx_refs[j], sem)
            for j in range(_NCH)
        ]
        copies += [
            pltpu.async_copy(field_hbm.at[row0 + j], val_refs[j], sem)
            for j in range(_NCH)
        ]

        def zfill(i, carry):
            zbuf[pl.ds(i * 16, 16)] = jnp.zeros((16,), jnp.float32)
            return carry

        lax.fori_loop(0, _ZCH // 16, zfill, 0)
        sl0 = s * _TSLICE
        for t in range(4):
            pltpu.sync_copy(zbuf, shared.at[pl.ds(sl0 + t * _ZCH, _ZCH)])
        pltpu.sync_copy(
            zbuf.at[pl.ds(0, _TSLICE - 4 * _ZCH)],
            shared.at[pl.ds(sl0 + 4 * _ZCH, _TSLICE - 4 * _ZCH)],
        )

        for cp in copies:
            cp.wait()

        for j in range(_NCH):
            for g in range(8):
                iv = idx_refs[j][pl.ds(g * 16, 16)]
                rel = iv - scbase
                ok = (rel >= 0) & (rel < _HALF0)
                rout_refs[j][pl.ds(g * 16, 16)] = jnp.where(ok, rel, _DUMP)

        plsc.subcore_barrier()

        scats = [
            pltpu.async_copy(val_refs[j], shared.at[rout_refs[j]], sem)
            for j in range(_NCH)
        ]
        for cp in scats:
            cp.wait()

        plsc.subcore_barrier()
        lax.fori_loop(0, _ZCH // 16, zfill, 0)

        obase = scbase + sl0
        for t in range(4):
            pltpu.sync_copy(shared.at[pl.ds(sl0 + t * _ZCH, _ZCH)], zbuf)
            pltpu.sync_copy(zbuf, out_hbm.at[pl.ds(obase + t * _ZCH, _ZCH)])
        tail = _TSLICE - 4 * _ZCH
        pltpu.sync_copy(
            shared.at[pl.ds(sl0 + 4 * _ZCH, tail)], zbuf.at[pl.ds(0, tail)]
        )
        pltpu.sync_copy(
            zbuf.at[pl.ds(0, tail)], out_hbm.at[pl.ds(obase + 4 * _ZCH, tail)]
        )

    return k(field2d, idx2d)


def kernel(states, site_flat_idx):
    idx2d = site_flat_idx.reshape(_SROW, 128)
    field2d = _compute_field(idx2d, states)
    return _sc_scatter(field2d, idx2d)[:_M]
